# baseline (speedup 1.0000x reference)
import functools

import jax
import jax.numpy as jnp
from jax import lax
from jax.experimental import pallas as pl
from jax.experimental.pallas import tpu as pltpu
from jax.experimental.pallas import tpu_sc as plsc


_SCALE = 32.0

_NC = 2
_NS = 16
_NW = _NC * _NS

_B = 4
_SEQ = 8192
_D = 1024
_ROWS_W = _SEQ // _NW
_R = 4
_NCHUNK = _ROWS_W // _R
_VPR = _D // 16
_NSET = 6


def _fma_chunk4(x4, pebuf):
    @plsc.parallel_loop(0, _R * _VPR, step=1, unroll=4)
    def body(k):
        i = k >> 6
        sl = pl.ds((k & (_VPR - 1)) * 16, 16)
        pe = pebuf[i, sl]
        for xb in x4:
            xb[i, sl] = xb[i, sl] * _SCALE + pe


def _sc_body(x_hbm, pe_hbm, out_hbm, *scratch):
    xbufs = scratch[0:_NSET * _B]
    pebufs = scratch[_NSET * _B:_NSET * _B + 2]
    lsems = scratch[_NSET * _B + 2:_NSET * _B + 2 + _NSET]
    ssems = scratch[_NSET * _B + 2 + _NSET:_NSET * _B + 2 + 2 * _NSET]
    spe = scratch[-1]

    wid = lax.axis_index("s") * _NC + lax.axis_index("c")
    row0 = wid * _ROWS_W

    def xset(k):
        return xbufs[_B * k:_B * (k + 1)]

    def issue_loads(c, k):
        for b in range(_B):
            pltpu.async_copy(
                x_hbm.at[b, pl.ds(row0 + c * _R, _R), :],
                xset(k)[b], lsems[k])

    def wait_loads(k):
        for b in range(_B):
            pltpu.make_async_copy(
                x_hbm.at[b, pl.ds(row0, _R), :], xset(k)[b],
                lsems[k]).wait()

    def issue_stores(c, k):
        for b in range(_B):
            pltpu.async_copy(
                xset(k)[b], out_hbm.at[b, pl.ds(row0 + c * _R, _R), :],
                ssems[k])

    def wait_stores(k):
        for b in range(_B):
            pltpu.make_async_copy(
                xset(k)[b], out_hbm.at[b, pl.ds(row0, _R), :],
                ssems[k]).wait()

    def issue_pe_kb(c, kb):
        pltpu.async_copy(
            pe_hbm.at[pl.ds(row0 + c * _R, _R), :], pebufs[kb], spe)

    def wait_pe(kb):
        pltpu.make_async_copy(
            pe_hbm.at[pl.ds(row0, _R), :], pebufs[kb], spe).wait()

    def slot(c, k, kb, *, pe_wait, pe_next, st_wait, ld_next):
        if pe_next:
            issue_pe_kb(c + 1, 1 - kb)
        if pe_wait:
            wait_pe(kb)
        wait_loads(k)
        _fma_chunk4(xset(k), pebufs[kb])
        issue_stores(c, k)
        if st_wait:
            wait_stores((k + 2) % _NSET)
        if ld_next:
            issue_loads(c + 2, (k + 2) % _NSET)

    pltpu.sync_copy(pe_hbm.at[pl.ds(row0, _R), :], pebufs[0])
    issue_loads(0, 0)
    issue_loads(1, 1)
    for c in range(_NSET - 2):
        slot(c, c, c % 2, pe_wait=(c > 0), pe_next=True, st_wait=False,
             ld_next=True)

    first = _NSET - 2
    n_groups = (_NCHUNK - first - _NSET) // _NSET

    def group(g, carry):
        c0 = first + g * _NSET
        for i in range(_NSET):
            slot(c0 + i, (first + i) % _NSET, i % 2,
                 pe_wait=True, pe_next=True, st_wait=True, ld_next=True)
        return carry
    lax.fori_loop(0, n_groups, group, 0)

    for c in range(_NCHUNK - _NSET, _NCHUNK):
        slot(c, c % _NSET, c % 2,
             pe_wait=True, pe_next=(c + 1 < _NCHUNK), st_wait=True,
             ld_next=(c + 2 < _NCHUNK))
    for c in range(_NCHUNK - _NSET + 2, _NCHUNK):
        wait_stores(c % _NSET)


def _sc_call(x, pos_emb):
    mesh = plsc.VectorSubcoreMesh(core_axis_name="c", subcore_axis_name="s")
    run = functools.partial(
        pl.kernel,
        mesh=mesh,
        out_type=jax.ShapeDtypeStruct((_B, _SEQ, _D), jnp.float32),
        scratch_types=(
            [pltpu.VMEM((_R, _D), jnp.float32)] * (_NSET * _B + 2)
            + [pltpu.SemaphoreType.DMA] * (2 * _NSET + 1)
        ),
    )(_sc_body)
    return run(x, pos_emb)


def kernel(x, pos_emb):
    b, s, d = x.shape
    assert (b, s, d) == (_B, _SEQ, _D), (b, s, d)
    return _sc_call(x, pos_emb[:s])


# speedup vs baseline: 1.0207x; 1.0207x over previous
import functools

import jax
import jax.numpy as jnp
from jax import lax
from jax.experimental import pallas as pl
from jax.experimental.pallas import tpu as pltpu
from jax.experimental.pallas import tpu_sc as plsc


_SCALE = 32.0

_NC = 2
_NS = 16
_NW = _NC * _NS

_B = 4
_SEQ = 8192
_D = 1024
_ROWS_W = _SEQ // _NW
_R = 4
_NCHUNK = _ROWS_W // _R
_VPR = _D // 16
_NSET = 4


def _fma_chunk4(x4, pebuf):
    @plsc.parallel_loop(0, _R * _VPR, step=1, unroll=4)
    def body(k):
        i = k >> 6
        sl = pl.ds((k & (_VPR - 1)) * 16, 16)
        pe = pebuf[i, sl]
        for xb in x4:
            xb[i, sl] = xb[i, sl] * _SCALE + pe


def _sc_body(x_hbm, pe_hbm, out_hbm, *scratch):
    xbufs = scratch[0:_NSET * _B]
    pebufs = scratch[_NSET * _B:_NSET * _B + 2]
    lsems = scratch[_NSET * _B + 2:_NSET * _B + 2 + _NSET]
    ssems = scratch[_NSET * _B + 2 + _NSET:_NSET * _B + 2 + 2 * _NSET]
    spe = scratch[-1]

    wid = lax.axis_index("s") * _NC + lax.axis_index("c")
    row0 = wid * _ROWS_W

    def xset(k):
        return xbufs[_B * k:_B * (k + 1)]

    def issue_loads(c, k):
        for b in range(_B):
            pltpu.async_copy(
                x_hbm.at[b, pl.ds(row0 + c * _R, _R), :],
                xset(k)[b], lsems[k])

    def wait_loads(k):
        for b in range(_B):
            pltpu.make_async_copy(
                x_hbm.at[b, pl.ds(row0, _R), :], xset(k)[b],
                lsems[k]).wait()

    def issue_stores(c, k):
        for b in range(_B):
            pltpu.async_copy(
                xset(k)[b], out_hbm.at[b, pl.ds(row0 + c * _R, _R), :],
                ssems[k])

    def wait_stores(k):
        for b in range(_B):
            pltpu.make_async_copy(
                xset(k)[b], out_hbm.at[b, pl.ds(row0, _R), :],
                ssems[k]).wait()

    def issue_pe_kb(c, kb):
        pltpu.async_copy(
            pe_hbm.at[pl.ds(row0 + c * _R, _R), :], pebufs[kb], spe)

    def wait_pe(kb):
        pltpu.make_async_copy(
            pe_hbm.at[pl.ds(row0, _R), :], pebufs[kb], spe).wait()

    def slot(c, k, kb, *, pe_wait, pe_next, st_wait, ld_next):
        if pe_next:
            issue_pe_kb(c + 1, 1 - kb)
        if pe_wait:
            wait_pe(kb)
        wait_loads(k)
        _fma_chunk4(xset(k), pebufs[kb])
        issue_stores(c, k)
        if st_wait:
            wait_stores((k + 2) % _NSET)
        if ld_next:
            issue_loads(c + 2, (k + 2) % _NSET)

    pltpu.sync_copy(pe_hbm.at[pl.ds(row0, _R), :], pebufs[0])
    issue_loads(0, 0)
    issue_loads(1, 1)
    for c in range(_NSET - 2):
        slot(c, c, c % 2, pe_wait=(c > 0), pe_next=True, st_wait=False,
             ld_next=True)

    first = _NSET - 2
    n_groups = (_NCHUNK - first - 2) // _NSET

    def group(g, carry):
        c0 = first + g * _NSET
        for i in range(_NSET):
            slot(c0 + i, (first + i) % _NSET, i % 2,
                 pe_wait=True, pe_next=True, st_wait=True, ld_next=True)
        return carry
    lax.fori_loop(0, n_groups, group, 0)

    for c in range(_NCHUNK - 2, _NCHUNK):
        slot(c, c % _NSET, c % 2,
             pe_wait=True, pe_next=(c + 1 < _NCHUNK), st_wait=True,
             ld_next=(c + 2 < _NCHUNK))
    for c in range(_NCHUNK - 2, _NCHUNK):
        wait_stores(c % _NSET)


def _sc_call(x, pos_emb):
    mesh = plsc.VectorSubcoreMesh(core_axis_name="c", subcore_axis_name="s")
    run = functools.partial(
        pl.kernel,
        mesh=mesh,
        out_type=jax.ShapeDtypeStruct((_B, _SEQ, _D), jnp.float32),
        scratch_types=(
            [pltpu.VMEM((_R, _D), jnp.float32)] * (_NSET * _B + 2)
            + [pltpu.SemaphoreType.DMA] * (2 * _NSET + 1)
        ),
    )(_sc_body)
    return run(x, pos_emb)


def kernel(x, pos_emb):
    b, s, d = x.shape
    assert (b, s, d) == (_B, _SEQ, _D), (b, s, d)
    return _sc_call(x, pos_emb[:s])


# speedup vs baseline: 1.0230x; 1.0022x over previous
import functools

import jax
import jax.numpy as jnp
from jax import lax
from jax.experimental import pallas as pl
from jax.experimental.pallas import tpu as pltpu
from jax.experimental.pallas import tpu_sc as plsc


_SCALE = 32.0

_NC = 2
_NS = 16
_NW = _NC * _NS

_B = 4
_SEQ = 8192
_D = 1024
_ROWS_W = _SEQ // _NW
_R = 4
_NCHUNK = _ROWS_W // _R
_VPR = _D // 16
_NSET = 4


def _fma_chunk4(x4, pebuf):
    @plsc.parallel_loop(0, _R * _VPR, step=1, unroll=2)
    def body(k):
        i = k >> 6
        sl = pl.ds((k & (_VPR - 1)) * 16, 16)
        pe = pebuf[i, sl]
        for xb in x4:
            xb[i, sl] = xb[i, sl] * _SCALE + pe


def _sc_body(x_hbm, pe_hbm, out_hbm, *scratch):
    xbufs = scratch[0:_NSET * _B]
    pebufs = scratch[_NSET * _B:_NSET * _B + 2]
    lsems = scratch[_NSET * _B + 2:_NSET * _B + 2 + _NSET]
    ssems = scratch[_NSET * _B + 2 + _NSET:_NSET * _B + 2 + 2 * _NSET]
    spe = scratch[-1]

    wid = lax.axis_index("s") * _NC + lax.axis_index("c")
    row0 = wid * _ROWS_W

    def xset(k):
        return xbufs[_B * k:_B * (k + 1)]

    def issue_loads(c, k):
        for b in range(_B):
            pltpu.async_copy(
                x_hbm.at[b, pl.ds(row0 + c * _R, _R), :],
                xset(k)[b], lsems[k])

    def wait_loads(k):
        for b in range(_B):
            pltpu.make_async_copy(
                x_hbm.at[b, pl.ds(row0, _R), :], xset(k)[b],
                lsems[k]).wait()

    def issue_stores(c, k):
        for b in range(_B):
            pltpu.async_copy(
                xset(k)[b], out_hbm.at[b, pl.ds(row0 + c * _R, _R), :],
                ssems[k])

    def wait_stores(k):
        for b in range(_B):
            pltpu.make_async_copy(
                xset(k)[b], out_hbm.at[b, pl.ds(row0, _R), :],
                ssems[k]).wait()

    def issue_pe_kb(c, kb):
        pltpu.async_copy(
            pe_hbm.at[pl.ds(row0 + c * _R, _R), :], pebufs[kb], spe)

    def wait_pe(kb):
        pltpu.make_async_copy(
            pe_hbm.at[pl.ds(row0, _R), :], pebufs[kb], spe).wait()

    def slot(c, k, kb, *, pe_wait, pe_next, st_wait, ld_next):
        if pe_next:
            issue_pe_kb(c + 1, 1 - kb)
        if pe_wait:
            wait_pe(kb)
        wait_loads(k)
        _fma_chunk4(xset(k), pebufs[kb])
        issue_stores(c, k)
        if st_wait:
            wait_stores((k + 2) % _NSET)
        if ld_next:
            issue_loads(c + 2, (k + 2) % _NSET)

    pltpu.sync_copy(pe_hbm.at[pl.ds(row0, _R), :], pebufs[0])
    issue_loads(0, 0)
    issue_loads(1, 1)
    for c in range(_NSET - 2):
        slot(c, c, c % 2, pe_wait=(c > 0), pe_next=True, st_wait=False,
             ld_next=True)

    first = _NSET - 2
    n_groups = (_NCHUNK - first - 2) // _NSET

    def group(g, carry):
        c0 = first + g * _NSET
        for i in range(_NSET):
            slot(c0 + i, (first + i) % _NSET, i % 2,
                 pe_wait=True, pe_next=True, st_wait=True, ld_next=True)
        return carry
    lax.fori_loop(0, n_groups, group, 0)

    for c in range(_NCHUNK - 2, _NCHUNK):
        slot(c, c % _NSET, c % 2,
             pe_wait=True, pe_next=(c + 1 < _NCHUNK), st_wait=True,
             ld_next=(c + 2 < _NCHUNK))
    for c in range(_NCHUNK - 2, _NCHUNK):
        wait_stores(c % _NSET)


def _sc_call(x, pos_emb):
    mesh = plsc.VectorSubcoreMesh(core_axis_name="c", subcore_axis_name="s")
    run = functools.partial(
        pl.kernel,
        mesh=mesh,
        out_type=jax.ShapeDtypeStruct((_B, _SEQ, _D), jnp.float32),
        scratch_types=(
            [pltpu.VMEM((_R, _D), jnp.float32)] * (_NSET * _B + 2)
            + [pltpu.SemaphoreType.DMA] * (2 * _NSET + 1)
        ),
    )(_sc_body)
    return run(x, pos_emb)


def kernel(x, pos_emb):
    b, s, d = x.shape
    assert (b, s, d) == (_B, _SEQ, _D), (b, s, d)
    return _sc_call(x, pos_emb[:s])
